# baseline (speedup 1.0000x reference)
import functools

import jax
import jax.numpy as jnp
import numpy as np
from jax import lax
from jax.experimental import pallas as pl
from jax.experimental.pallas import tpu as pltpu
from jax.experimental.pallas import tpu_sc as plsc

N_ATOMS = 32768
MAX_NBR = 12
NBR_FEA_LEN = 64
HID_DIM = 768
MAX_GRAPH_LEN = 300
B = 16
L = 2048

NW = 32
ROWS = B * MAX_GRAPH_LEN
CH = 8
POS_PAD = 320
CPC = POS_PAD // CH
NCHUNK = B * CPC
CPW = NCHUNK // NW
NTYPE = 119
EBLK_ROWS = N_ATOMS // 8
APT = N_ATOMS // NW

_THREEFRY_ROTATIONS = ((13, 15, 26, 6), (17, 29, 16, 24))


def _threefry2x32(k0, k1, x0, x1):
    ks = (k0, k1, k0 ^ k1 ^ np.uint32(0x1BD11BDA))
    x0 = x0 + ks[0]
    x1 = x1 + ks[1]
    for i in range(5):
        for r in _THREEFRY_ROTATIONS[i % 2]:
            x0 = x0 + x1
            x1 = (x1 << np.uint32(r)) | (x1 >> np.uint32(32 - r))
            x1 = x0 ^ x1
        x0 = x0 + ks[(i + 1) % 3]
        x1 = x1 + ks[(i + 1) % 3] + np.uint32(i + 1)
    return x0, x1


def _prefix_bits(subkey, n):
    if jax.config.jax_threefry_partitionable:
        return jax.random.bits(subkey, (L,), jnp.uint32)
    kd = jax.random.key_data(subkey).astype(jnp.uint32)
    half = L // 2
    pos = jnp.arange(L)
    j = jnp.arange(half, dtype=jnp.uint32)
    n32 = jnp.asarray(n, jnp.uint32)
    m = (n32 + (n32 & jnp.uint32(1))) // jnp.uint32(2)
    x1 = jnp.where(j + m < n32, j + m, jnp.uint32(0))
    o0, o1 = _threefry2x32(kd[0], kd[1], j, x1)
    mi = m.astype(pos.dtype)
    idx0 = jnp.clip(pos, 0, half - 1)
    idx1 = jnp.clip(pos - mi, 0, half - 1)
    return jnp.where(pos < mi, o0[idx0], o1[idx1])


def _subkeys(key):
    key1, sub1 = jax.random.split(key)
    _, sub2 = jax.random.split(key1)
    return sub1, sub2


def _build_positions(n_others, n_carbon):
    ns = jnp.stack([n_others, n_carbon], axis=1).reshape(-1)
    perm_base = jax.random.key(1)
    keys = jax.vmap(lambda i: jax.random.fold_in(perm_base, i))(jnp.arange(2 * B))
    sub1, sub2 = jax.vmap(_subkeys)(keys)
    bits1 = jax.vmap(_prefix_bits)(sub1, ns)
    bits2 = jax.vmap(_prefix_bits)(sub2, ns)
    pos = jnp.arange(L)
    sentinel = jnp.uint32(0xFFFFFFFF)
    msk = pos[None, :] < ns[:, None]
    k1 = jnp.where(msk, bits1, sentinel)
    k2 = jnp.where(msk, bits2, sentinel)
    vals = jnp.broadcast_to(jnp.arange(L), (4 * B, L))
    _, vs = jax.lax.sort_key_val(jnp.concatenate([k1, k2], axis=0), vals)
    v1 = vs[:2 * B]
    w2 = vs[2 * B:, :180]
    v2 = jnp.take_along_axis(v1, w2, axis=1)
    two_round = ns > int(np.iinfo(np.uint32).max ** (1.0 / 3.0))
    pref = jnp.where(two_round[:, None], v2, v1[:, :180])
    perm_o = pref[0::2, :180]
    perm_c = pref[1::2, :120]
    sel_pos = jnp.concatenate([perm_o, perm_c], axis=1)
    return jnp.pad(sel_pos, ((0, 0), (0, POS_PAD - MAX_GRAPH_LEN))
                   ).reshape(B * POS_PAD).astype(jnp.int32)


@functools.lru_cache(maxsize=1)
def _make_count_kernel():
    mesh = plsc.VectorSubcoreMesh(core_axis_name="c", subcore_axis_name="s",
                                  num_cores=2, num_subcores=16)

    @functools.partial(
        pl.kernel,
        out_type=(jax.ShapeDtypeStruct((NW * 16,), jnp.int32),
                  jax.ShapeDtypeStruct((EBLK_ROWS, 128), jnp.int32)),
        mesh=mesh,
        compiler_params=pltpu.CompilerParams(needs_layout_passes=False),
        scratch_types=[
            pltpu.VMEM((N_ATOMS,), jnp.int32),
            pltpu.VMEM((APT,), jnp.int32),
            pltpu.VMEM((APT * MAX_NBR,), jnp.int32),
            pltpu.VMEM((APT // 8, 128), jnp.int32),
            pltpu.VMEM((16,), jnp.int32),
        ],
    )
    def count_kernel(anum_h, cai_h, nbr_h, cnt_h, eblk_h,
                     anum_v, cai_v, nbr_v, eout_v, res_v):
        wid = lax.axis_index("s") * 2 + lax.axis_index("c")
        pltpu.sync_copy(anum_h, anum_v)
        pltpu.sync_copy(cai_h.at[pl.ds(wid * APT, APT)], cai_v)
        pltpu.sync_copy(nbr_h.at[pl.ds(wid * APT * MAX_NBR, APT * MAX_NBR)],
                        nbr_v)

        iota = lax.iota(jnp.int32, 16)
        col = jnp.minimum(iota, MAX_NBR - 1)

        def ebody(g, carry):
            for t in range(4):
                a = g * 4 + t
                nids = plsc.load_gather(nbr_v, [a * MAX_NBR + col])
                self_id = jnp.full((16,), wid * APT + a, jnp.int32)
                nids = jnp.where(iota < MAX_NBR, nids, self_id)
                ei = plsc.load_gather(anum_v, [nids])
                eout_v[lax.shift_right_logical(a, 3),
                       pl.ds((a & 7) * 16, 16)] = ei
            return carry

        lax.fori_loop(0, APT // 4, ebody, 0)
        pltpu.sync_copy(eout_v, eblk_h.at[pl.ds(wid * (APT // 8), APT // 8)])

        def body(v, accs):
            acc_o, acc_c = accs
            an = plsc.load_gather(anum_v, [cai_v[pl.ds(v * 16, 16)]])
            one = jnp.ones((16,), jnp.int32)
            zero = jnp.zeros((16,), jnp.int32)
            is_c = an == 6
            is_o = jnp.logical_and(an != 6, an != 1)
            return (acc_o + jnp.where(is_o, one, zero),
                    acc_c + jnp.where(is_c, one, zero))

        acc_o, acc_c = lax.fori_loop(
            0, APT // 16, body,
            (jnp.zeros((16,), jnp.int32), jnp.zeros((16,), jnp.int32)))
        so = jnp.sum(acc_o)
        sc = jnp.sum(acc_c)
        res = jnp.where(iota == 0, jnp.full((16,), so, jnp.int32),
                        jnp.where(iota == 1, jnp.full((16,), sc, jnp.int32),
                                  jnp.zeros((16,), jnp.int32)))
        res_v[...] = res
        pltpu.sync_copy(res_v, cnt_h.at[pl.ds(wid * 16, 16)])

    return count_kernel


@functools.lru_cache(maxsize=1)
def _make_main_kernel():
    mesh = plsc.VectorSubcoreMesh(core_axis_name="c", subcore_axis_name="s",
                                  num_cores=2, num_subcores=16)

    @functools.partial(
        pl.kernel,
        out_type=jax.ShapeDtypeStruct((B, POS_PAD, HID_DIM), jnp.float32),
        mesh=mesh,
        compiler_params=pltpu.CompilerParams(needs_layout_passes=False),
        scratch_types=[
            pltpu.VMEM((NTYPE, NBR_FEA_LEN), jnp.float32),
            pltpu.VMEM((NTYPE, NBR_FEA_LEN), jnp.float32),
            pltpu.VMEM((32,), jnp.int32),
            pltpu.VMEM((176,), jnp.int32),
            pltpu.VMEM((16,), jnp.int32),
            pltpu.VMEM((16,), jnp.int32),
            pltpu.VMEM((16, 128), jnp.int32),
            pltpu.VMEM((16, 128), jnp.int32),
            pltpu.VMEM((CH, HID_DIM), jnp.float32),
            pltpu.VMEM((CH, HID_DIM), jnp.float32),
            pltpu.SemaphoreType.DMA,
            pltpu.SemaphoreType.DMA,
            pltpu.SemaphoreType.DMA,
            pltpu.SemaphoreType.DMA,
        ] + [pltpu.VMEM((MAX_NBR, NBR_FEA_LEN), jnp.float32)] * (2 * CH),
    )
    def main_kernel(eblk_h, fea_h, node_h, edge_h, aidx_h, nn_h,
                    out_h, node_v, edge_v, nn_v, aall_v,
                    arow_a, arow_b, eb_a, eb_b, buf_a, buf_b,
                    semf_a, seme_a, semf_b, seme_b, *slabs):
        wid = lax.axis_index("s") * 2 + lax.axis_index("c")
        pltpu.sync_copy(node_h, node_v)
        pltpu.sync_copy(edge_h, edge_v)
        pltpu.sync_copy(nn_h, nn_v)
        pltpu.sync_copy(aidx_h.at[pl.ds(wid * 176, 176)], aall_v)

        iota = lax.iota(jnp.int32, 16)

        slots = ((arow_a, eb_a, slabs[:CH], buf_a, semf_a, seme_a),
                 (arow_b, eb_b, slabs[CH:], buf_b, semf_b, seme_b))

        def issue(k, slot):
            arow_v, eb_v, slab_l, buf_v, semf, seme = slot
            a = aall_v[pl.ds(k * CH, 16)]
            arow_v[...] = lax.shift_right_logical(a, 3)
            pltpu.async_copy(eblk_h.at[arow_v], eb_v, seme)
            for r in range(CH):
                a_s = jnp.sum(jnp.where(iota == r, a, 0))
                pltpu.async_copy(fea_h.at[a_s], slab_l[r], semf)

        def process(k, slot):
            arow_v, eb_v, slab_l, buf_v, semf, seme = slot
            c = wid + k * NW
            bi = lax.div(c, CPC)
            jc = c - bi * CPC
            j0 = jc * CH
            bi_spl = jnp.full((16,), bi, jnp.int32)
            no = plsc.load_gather(nn_v, [bi_spl])
            nc = plsc.load_gather(nn_v, [bi_spl + 16])
            pltpu.make_async_copy(eblk_h.at[arow_v], eb_v, seme).wait()
            for r in range(CH):
                pltpu.make_async_copy(fea_h.at[0], slab_l[r], semf).wait()

            for r in range(CH):
                r_spl = jnp.full((16,), r, jnp.int32)
                a_spl = plsc.load_gather(aall_v, [r_spl + k * CH])
                ecol = (a_spl & 7) * 16
                na_spl = plsc.load_gather(eb_v, [r_spl, ecol + MAX_NBR])
                j_spl = jnp.full((16,), j0 + r, jnp.int32)
                ok = jnp.where(j_spl < 180, j_spl < no, j_spl - 180 < nc)
                val_spl = jnp.where(ok, jnp.full((16,), 1.0, jnp.float32),
                                    jnp.zeros((16,), jnp.float32))
                nd = [plsc.load_gather(node_v, [na_spl, iota + q * 16])
                      for q in range(4)]
                for m in range(MAX_NBR):
                    ei_spl = plsc.load_gather(eb_v, [r_spl, ecol + m])
                    m_spl = jnp.full((16,), m, jnp.int32)
                    for q in range(4):
                        ed = plsc.load_gather(edge_v, [ei_spl, iota + q * 16])
                        fe = plsc.load_gather(slab_l[r],
                                              [m_spl, iota + q * 16])
                        buf_v[r, pl.ds(m * NBR_FEA_LEN + q * 16, 16)] = (
                            (nd[q] + ed + fe) * val_spl)

            pltpu.sync_copy(buf_v, out_h.at[bi, pl.ds(j0, CH)])

        issue(0, slots[0])

        def pair_body(k2, carry):
            issue(2 * k2 + 1, slots[1])
            process(2 * k2, slots[0])

            @pl.when(k2 < CPW // 2 - 1)
            def _issue_next():
                issue(2 * k2 + 2, slots[0])

            process(2 * k2 + 1, slots[1])
            return carry

        lax.fori_loop(0, CPW // 2, pair_body, 0)

    return main_kernel


def kernel(atom_num, nbr_idx, nbr_fea, crystal_atom_idx, node_table, edge_table):
    atom_num = atom_num.astype(jnp.int32)
    nbr_idx = nbr_idx.astype(jnp.int32)
    cai_flat = crystal_atom_idx.astype(jnp.int32).reshape(-1)

    cnts, eblk = _make_count_kernel()(atom_num, cai_flat,
                                      nbr_idx.reshape(-1))
    parts = cnts.reshape(NW, 16)
    n_others = parts[0::2, 0] + parts[1::2, 0]
    n_carbon = parts[0::2, 1] + parts[1::2, 1]

    pos = _build_positions(n_others, n_carbon)
    nn = jnp.concatenate([n_others, n_carbon]).astype(jnp.int32)

    sel_pos = pos.reshape(B, POS_PAD)
    asel = jnp.take_along_axis(crystal_atom_idx.astype(jnp.int32),
                               sel_pos, axis=1)
    wm = asel.reshape(NCHUNK, CH).reshape(CPW, NW, CH).transpose(
        1, 0, 2).reshape(NW, CPW * CH)
    aidx_wm = jnp.pad(wm, ((0, 0), (0, 176 - CPW * CH))).reshape(-1)

    out = _make_main_kernel()(eblk, nbr_fea, node_table, edge_table,
                              aidx_wm, nn)
    graph_emb = out[:, :MAX_GRAPH_LEN]
    mask = (graph_emb.sum(axis=-1) != 0).astype(jnp.float32)
    return graph_emb, mask


# speedup vs baseline: 1.5505x; 1.5505x over previous
import functools

import jax
import jax.numpy as jnp
import numpy as np
from jax import lax
from jax.experimental import pallas as pl
from jax.experimental.pallas import tpu as pltpu
from jax.experimental.pallas import tpu_sc as plsc

N_ATOMS = 32768
MAX_NBR = 12
NBR_FEA_LEN = 64
HID_DIM = 768
MAX_GRAPH_LEN = 300
B = 16
L = 2048

NW = 32
ROWS = B * MAX_GRAPH_LEN
CH = 16
POS_PAD = 304
CPC = POS_PAD // CH
NCHUNK = B * CPC
NTYPE = 119
EBLK_ROWS = N_ATOMS // 8
APT = N_ATOMS // NW

_THREEFRY_ROTATIONS = ((13, 15, 26, 6), (17, 29, 16, 24))


def _threefry2x32(k0, k1, x0, x1):
    ks = (k0, k1, k0 ^ k1 ^ np.uint32(0x1BD11BDA))
    x0 = x0 + ks[0]
    x1 = x1 + ks[1]
    for i in range(5):
        for r in _THREEFRY_ROTATIONS[i % 2]:
            x0 = x0 + x1
            x1 = (x1 << np.uint32(r)) | (x1 >> np.uint32(32 - r))
            x1 = x0 ^ x1
        x0 = x0 + ks[(i + 1) % 3]
        x1 = x1 + ks[(i + 1) % 3] + np.uint32(i + 1)
    return x0, x1


def _prefix_bits(subkey, n):
    if jax.config.jax_threefry_partitionable:
        return jax.random.bits(subkey, (L,), jnp.uint32)
    kd = jax.random.key_data(subkey).astype(jnp.uint32)
    half = L // 2
    pos = jnp.arange(L)
    j = jnp.arange(half, dtype=jnp.uint32)
    n32 = jnp.asarray(n, jnp.uint32)
    m = (n32 + (n32 & jnp.uint32(1))) // jnp.uint32(2)
    x1 = jnp.where(j + m < n32, j + m, jnp.uint32(0))
    o0, o1 = _threefry2x32(kd[0], kd[1], j, x1)
    mi = m.astype(pos.dtype)
    idx0 = jnp.clip(pos, 0, half - 1)
    idx1 = jnp.clip(pos - mi, 0, half - 1)
    return jnp.where(pos < mi, o0[idx0], o1[idx1])


def _subkeys(key):
    key1, sub1 = jax.random.split(key)
    _, sub2 = jax.random.split(key1)
    return sub1, sub2


def _build_positions(n_others, n_carbon):
    ns = jnp.stack([n_others, n_carbon], axis=1).reshape(-1)
    perm_base = jax.random.key(1)
    keys = jax.vmap(lambda i: jax.random.fold_in(perm_base, i))(jnp.arange(2 * B))
    sub1, sub2 = jax.vmap(_subkeys)(keys)
    bits1 = jax.vmap(_prefix_bits)(sub1, ns)
    bits2 = jax.vmap(_prefix_bits)(sub2, ns)
    pos = jnp.arange(L)
    sentinel = jnp.uint32(0xFFFFFFFF)
    msk = pos[None, :] < ns[:, None]
    k1 = jnp.where(msk, bits1, sentinel)
    k2 = jnp.where(msk, bits2, sentinel)
    vals = jnp.broadcast_to(jnp.arange(L), (4 * B, L))
    _, vs = jax.lax.sort_key_val(jnp.concatenate([k1, k2], axis=0), vals)
    v1 = vs[:2 * B]
    w2 = vs[2 * B:, :180]
    v2 = jnp.take_along_axis(v1, w2, axis=1)
    two_round = ns > int(np.iinfo(np.uint32).max ** (1.0 / 3.0))
    pref = jnp.where(two_round[:, None], v2, v1[:, :180])
    perm_o = pref[0::2, :180]
    perm_c = pref[1::2, :120]
    sel_pos = jnp.concatenate([perm_o, perm_c], axis=1)
    return jnp.pad(sel_pos, ((0, 0), (0, POS_PAD - MAX_GRAPH_LEN))
                   ).reshape(B * POS_PAD).astype(jnp.int32)


@functools.lru_cache(maxsize=1)
def _make_count_kernel():
    mesh = plsc.VectorSubcoreMesh(core_axis_name="c", subcore_axis_name="s",
                                  num_cores=2, num_subcores=16)

    @functools.partial(
        pl.kernel,
        out_type=(jax.ShapeDtypeStruct((NW * 16,), jnp.int32),
                  jax.ShapeDtypeStruct((EBLK_ROWS, 128), jnp.int32)),
        mesh=mesh,
        compiler_params=pltpu.CompilerParams(needs_layout_passes=False),
        scratch_types=[
            pltpu.VMEM((N_ATOMS,), jnp.int32),
            pltpu.VMEM((APT,), jnp.int32),
            pltpu.VMEM((APT * MAX_NBR,), jnp.int32),
            pltpu.VMEM((APT // 8, 128), jnp.int32),
            pltpu.VMEM((16,), jnp.int32),
        ],
    )
    def count_kernel(anum_h, cai_h, nbr_h, cnt_h, eblk_h,
                     anum_v, cai_v, nbr_v, eout_v, res_v):
        wid = lax.axis_index("s") * 2 + lax.axis_index("c")
        pltpu.sync_copy(anum_h, anum_v)
        pltpu.sync_copy(cai_h.at[pl.ds(wid * APT, APT)], cai_v)
        pltpu.sync_copy(nbr_h.at[pl.ds(wid * APT * MAX_NBR, APT * MAX_NBR)],
                        nbr_v)

        iota = lax.iota(jnp.int32, 16)
        col = jnp.minimum(iota, MAX_NBR - 1)

        def ebody(g, carry):
            for t in range(4):
                a = g * 4 + t
                nids = plsc.load_gather(nbr_v, [a * MAX_NBR + col])
                self_id = jnp.full((16,), wid * APT + a, jnp.int32)
                nids = jnp.where(iota < MAX_NBR, nids, self_id)
                ei = plsc.load_gather(anum_v, [nids])
                eout_v[lax.shift_right_logical(a, 3),
                       pl.ds((a & 7) * 16, 16)] = ei
            return carry

        lax.fori_loop(0, APT // 4, ebody, 0)
        pltpu.sync_copy(eout_v, eblk_h.at[pl.ds(wid * (APT // 8), APT // 8)])

        def body(v, accs):
            acc_o, acc_c = accs
            an = plsc.load_gather(anum_v, [cai_v[pl.ds(v * 16, 16)]])
            one = jnp.ones((16,), jnp.int32)
            zero = jnp.zeros((16,), jnp.int32)
            is_c = an == 6
            is_o = jnp.logical_and(an != 6, an != 1)
            return (acc_o + jnp.where(is_o, one, zero),
                    acc_c + jnp.where(is_c, one, zero))

        acc_o, acc_c = lax.fori_loop(
            0, APT // 16, body,
            (jnp.zeros((16,), jnp.int32), jnp.zeros((16,), jnp.int32)))
        so = jnp.sum(acc_o)
        sc = jnp.sum(acc_c)
        res = jnp.where(iota == 0, jnp.full((16,), so, jnp.int32),
                        jnp.where(iota == 1, jnp.full((16,), sc, jnp.int32),
                                  jnp.zeros((16,), jnp.int32)))
        res_v[...] = res
        pltpu.sync_copy(res_v, cnt_h.at[pl.ds(wid * 16, 16)])

    return count_kernel


@functools.lru_cache(maxsize=1)
def _make_main_kernel():
    mesh = plsc.VectorSubcoreMesh(core_axis_name="c", subcore_axis_name="s",
                                  num_cores=2, num_subcores=16)

    @functools.partial(
        pl.kernel,
        out_type=jax.ShapeDtypeStruct((B, POS_PAD, HID_DIM), jnp.float32),
        mesh=mesh,
        compiler_params=pltpu.CompilerParams(needs_layout_passes=False),
        scratch_types=[
            pltpu.VMEM((N_ATOMS,), jnp.int32),
            pltpu.VMEM((NTYPE, NBR_FEA_LEN), jnp.float32),
            pltpu.VMEM((NTYPE, NBR_FEA_LEN), jnp.float32),
            pltpu.VMEM((32,), jnp.int32),
            pltpu.VMEM((160,), jnp.int32),
            pltpu.VMEM((16,), jnp.int32),
            pltpu.VMEM((16,), jnp.int32),
            pltpu.VMEM((16,), jnp.int32),
            pltpu.VMEM((16,), jnp.int32),
            pltpu.VMEM((CH, 128), jnp.int32),
            pltpu.VMEM((CH, 128), jnp.int32),
            pltpu.VMEM((CH, HID_DIM), jnp.float32),
            pltpu.VMEM((CH, HID_DIM), jnp.float32),
            pltpu.SemaphoreType.DMA,
            pltpu.SemaphoreType.DMA,
            pltpu.SemaphoreType.DMA,
            pltpu.SemaphoreType.DMA,
        ],
    )
    def main_kernel(cai_h, eblk_h, fea_h, node_h, edge_h, pos_h, nn_h,
                    out_h, cai_v, node_v, edge_v, nn_v, posall_v,
                    aidx_a, arow_a, aidx_b, arow_b, eb_a, eb_b, buf_a, buf_b,
                    semf_a, seme_a, semf_b, seme_b):
        wid = lax.axis_index("s") * 2 + lax.axis_index("c")
        pltpu.sync_copy(cai_h, cai_v)
        pltpu.sync_copy(node_h, node_v)
        pltpu.sync_copy(edge_h, edge_v)
        pltpu.sync_copy(nn_h, nn_v)
        pltpu.sync_copy(pos_h.at[pl.ds(wid * 160, 160)], posall_v)

        iota = lax.iota(jnp.int32, 16)
        nchunks = jnp.where(wid < NCHUNK - (NCHUNK // NW) * NW,
                            NCHUNK // NW + 1, NCHUNK // NW)

        slots = ((aidx_a, arow_a, eb_a, buf_a, semf_a, seme_a),
                 (aidx_b, arow_b, eb_b, buf_b, semf_b, seme_b))

        def issue(k, slot):
            aidx_v, arow_v, eb_v, buf_v, semf, seme = slot
            c = wid + k * NW
            bi = lax.div(c, CPC)
            a = plsc.load_gather(cai_v, [bi * L + posall_v[pl.ds(k * CH, CH)]])
            aidx_v[...] = a
            arow_v[...] = lax.shift_right_logical(a, 3)
            pltpu.async_copy(fea_h.at[aidx_v], buf_v, semf)
            pltpu.async_copy(eblk_h.at[arow_v], eb_v, seme)

        def process(k, slot):
            aidx_v, arow_v, eb_v, buf_v, semf, seme = slot
            c = wid + k * NW
            bi = lax.div(c, CPC)
            jc = c - bi * CPC
            j0 = jc * CH
            bi_spl = jnp.full((16,), bi, jnp.int32)
            no = plsc.load_gather(nn_v, [bi_spl])
            nc = plsc.load_gather(nn_v, [bi_spl + 16])
            pltpu.make_async_copy(fea_h.at[aidx_v], buf_v, semf).wait()
            pltpu.make_async_copy(eblk_h.at[arow_v], eb_v, seme).wait()

            def row_body(r, carry2):
                r_spl = jnp.full((16,), r, jnp.int32)
                a_spl = plsc.load_gather(aidx_v, [r_spl])
                ecol = (a_spl & 7) * 16
                na_spl = plsc.load_gather(eb_v, [r_spl, ecol + MAX_NBR])
                j_spl = jnp.full((16,), j0 + r, jnp.int32)
                ok = jnp.where(j_spl < 180, j_spl < no, j_spl - 180 < nc)
                val_spl = jnp.where(ok, jnp.full((16,), 1.0, jnp.float32),
                                    jnp.zeros((16,), jnp.float32))
                nd = [plsc.load_gather(node_v, [na_spl, iota + q * 16])
                      for q in range(4)]
                for m in range(MAX_NBR):
                    ei_spl = plsc.load_gather(eb_v, [r_spl, ecol + m])
                    for q in range(4):
                        ed = plsc.load_gather(edge_v, [ei_spl, iota + q * 16])
                        fe = buf_v[r, pl.ds(m * NBR_FEA_LEN + q * 16, 16)]
                        buf_v[r, pl.ds(m * NBR_FEA_LEN + q * 16, 16)] = (
                            (nd[q] + ed + fe) * val_spl)
                return carry2

            lax.fori_loop(0, CH, row_body, 0)
            pltpu.sync_copy(buf_v, out_h.at[bi, pl.ds(j0, CH)])

        issue(0, slots[0])

        def pair_body(k2, carry):
            k_b = 2 * k2 + 1
            k_a2 = 2 * k2 + 2

            @pl.when(k_b < nchunks)
            def _issue_b():
                issue(k_b, slots[1])

            process(2 * k2, slots[0])

            @pl.when(k_a2 < nchunks)
            def _issue_a():
                issue(k_a2, slots[0])

            @pl.when(k_b < nchunks)
            def _process_b():
                process(k_b, slots[1])

            return carry

        lax.fori_loop(0, (NCHUNK // NW + 2) // 2, pair_body, 0)

    return main_kernel


def kernel(atom_num, nbr_idx, nbr_fea, crystal_atom_idx, node_table, edge_table):
    atom_num = atom_num.astype(jnp.int32)
    nbr_idx = nbr_idx.astype(jnp.int32)
    cai_flat = crystal_atom_idx.astype(jnp.int32).reshape(-1)

    cnts, eblk = _make_count_kernel()(atom_num, cai_flat,
                                      nbr_idx.reshape(-1))
    parts = cnts.reshape(NW, 16)
    n_others = parts[0::2, 0] + parts[1::2, 0]
    n_carbon = parts[0::2, 1] + parts[1::2, 1]

    pos = _build_positions(n_others, n_carbon)
    nn = jnp.concatenate([n_others, n_carbon]).astype(jnp.int32)

    pos_chunks = jnp.pad(pos.reshape(NCHUNK, CH), ((0, 320 - NCHUNK), (0, 0)))
    pos_wm = pos_chunks.reshape(10, NW, CH).transpose(1, 0, 2).reshape(-1)

    fea2 = nbr_fea.reshape(N_ATOMS, HID_DIM)
    out = _make_main_kernel()(cai_flat, eblk, fea2, node_table, edge_table,
                              pos_wm, nn)
    graph_emb = out[:, :MAX_GRAPH_LEN]
    mask = (graph_emb.sum(axis=-1) != 0).astype(jnp.float32)
    return graph_emb, mask


# speedup vs baseline: 1.6338x; 1.0537x over previous
import functools

import jax
import jax.numpy as jnp
import numpy as np
from jax import lax
from jax.experimental import pallas as pl
from jax.experimental.pallas import tpu as pltpu
from jax.experimental.pallas import tpu_sc as plsc

N_ATOMS = 32768
MAX_NBR = 12
NBR_FEA_LEN = 64
HID_DIM = 768
MAX_GRAPH_LEN = 300
B = 16
L = 2048

NW = 32
ROWS = B * MAX_GRAPH_LEN
CH = 16
POS_PAD = 304
CPC = POS_PAD // CH
NCHUNK = B * CPC
NTYPE = 119
EBLK_ROWS = N_ATOMS // 8
APT = N_ATOMS // NW

_THREEFRY_ROTATIONS = ((13, 15, 26, 6), (17, 29, 16, 24))


def _threefry2x32(k0, k1, x0, x1):
    ks = (k0, k1, k0 ^ k1 ^ np.uint32(0x1BD11BDA))
    x0 = x0 + ks[0]
    x1 = x1 + ks[1]
    for i in range(5):
        for r in _THREEFRY_ROTATIONS[i % 2]:
            x0 = x0 + x1
            x1 = (x1 << np.uint32(r)) | (x1 >> np.uint32(32 - r))
            x1 = x0 ^ x1
        x0 = x0 + ks[(i + 1) % 3]
        x1 = x1 + ks[(i + 1) % 3] + np.uint32(i + 1)
    return x0, x1


def _prefix_bits(subkey, n):
    if jax.config.jax_threefry_partitionable:
        return jax.random.bits(subkey, (L,), jnp.uint32)
    kd = jax.random.key_data(subkey).astype(jnp.uint32)
    half = L // 2
    pos = jnp.arange(L)
    j = jnp.arange(half, dtype=jnp.uint32)
    n32 = jnp.asarray(n, jnp.uint32)
    m = (n32 + (n32 & jnp.uint32(1))) // jnp.uint32(2)
    x1 = jnp.where(j + m < n32, j + m, jnp.uint32(0))
    o0, o1 = _threefry2x32(kd[0], kd[1], j, x1)
    mi = m.astype(pos.dtype)
    idx0 = jnp.clip(pos, 0, half - 1)
    idx1 = jnp.clip(pos - mi, 0, half - 1)
    return jnp.where(pos < mi, o0[idx0], o1[idx1])


def _subkeys(key):
    key1, sub1 = jax.random.split(key)
    _, sub2 = jax.random.split(key1)
    return sub1, sub2


def _build_positions(n_others, n_carbon):
    ns = jnp.stack([n_others, n_carbon], axis=1).reshape(-1)
    perm_base = jax.random.key(1)
    keys = jax.vmap(lambda i: jax.random.fold_in(perm_base, i))(jnp.arange(2 * B))
    sub1, sub2 = jax.vmap(_subkeys)(keys)
    bits1 = jax.vmap(_prefix_bits)(sub1, ns)
    bits2 = jax.vmap(_prefix_bits)(sub2, ns)
    pos = jnp.arange(L)
    sentinel = jnp.uint32(0xFFFFFFFF)
    msk = pos[None, :] < ns[:, None]
    k1 = jnp.where(msk, bits1, sentinel)
    k2 = jnp.where(msk, bits2, sentinel)
    vals = jnp.broadcast_to(jnp.arange(L), (4 * B, L))
    _, vs = jax.lax.sort_key_val(jnp.concatenate([k1, k2], axis=0), vals)
    v1 = vs[:2 * B]
    w2 = vs[2 * B:, :180]
    v2 = jnp.take_along_axis(v1, w2, axis=1)
    two_round = ns > int(np.iinfo(np.uint32).max ** (1.0 / 3.0))
    pref = jnp.where(two_round[:, None], v2, v1[:, :180])
    perm_o = pref[0::2, :180]
    perm_c = pref[1::2, :120]
    sel_pos = jnp.concatenate([perm_o, perm_c], axis=1)
    return jnp.pad(sel_pos, ((0, 0), (0, POS_PAD - MAX_GRAPH_LEN))
                   ).reshape(B * POS_PAD).astype(jnp.int32)


@functools.lru_cache(maxsize=1)
def _make_count_kernel():
    mesh = plsc.VectorSubcoreMesh(core_axis_name="c", subcore_axis_name="s",
                                  num_cores=2, num_subcores=16)

    @functools.partial(
        pl.kernel,
        out_type=(jax.ShapeDtypeStruct((NW * 16,), jnp.int32),
                  jax.ShapeDtypeStruct((EBLK_ROWS, 128), jnp.int32)),
        mesh=mesh,
        compiler_params=pltpu.CompilerParams(needs_layout_passes=False),
        scratch_types=[
            pltpu.VMEM((N_ATOMS,), jnp.int32),
            pltpu.VMEM((APT,), jnp.int32),
            pltpu.VMEM((64, MAX_NBR), jnp.int32),
            pltpu.VMEM((64, MAX_NBR), jnp.int32),
            pltpu.VMEM((APT // 8, 128), jnp.int32),
            pltpu.VMEM((16,), jnp.int32),
            pltpu.SemaphoreType.DMA,
            pltpu.SemaphoreType.DMA,
        ],
    )
    def count_kernel(anum_h, cai_h, nbr_h, cnt_h, eblk_h,
                     anum_v, cai_v, slab_a, slab_b, eout_v, res_v,
                     sem_a, sem_b):
        wid = lax.axis_index("s") * 2 + lax.axis_index("c")
        pltpu.sync_copy(anum_h, anum_v)
        pltpu.sync_copy(cai_h.at[pl.ds(wid * APT, APT)], cai_v)

        iota = lax.iota(jnp.int32, 16)
        col = jnp.minimum(iota, MAX_NBR - 1)
        nslab = APT // 64
        slabs = ((slab_a, sem_a), (slab_b, sem_b))

        def nbr_rows(g):
            return nbr_h.at[pl.ds(wid * APT + g * 64, 64)]

        def eslab(g, slab_v, sem):
            pltpu.make_async_copy(nbr_rows(g), slab_v, sem).wait()
            base = g * 64

            def ebody(t, carry):
                for u in range(4):
                    rel = t * 4 + u
                    a = base + rel
                    nids = plsc.load_gather(
                        slab_v, [jnp.full((16,), rel, jnp.int32), col])
                    self_id = jnp.full((16,), wid * APT + a, jnp.int32)
                    nids = jnp.where(iota < MAX_NBR, nids, self_id)
                    ei = plsc.load_gather(anum_v, [nids])
                    eout_v[lax.shift_right_logical(a, 3),
                           pl.ds((a & 7) * 16, 16)] = ei
                return carry

            lax.fori_loop(0, 16, ebody, 0)

        pltpu.async_copy(nbr_rows(0), slab_a, sem_a)

        def spair(g2, carry):
            pltpu.async_copy(nbr_rows(2 * g2 + 1), slab_b, sem_b)
            eslab(2 * g2, slab_a, sem_a)

            @pl.when(g2 < nslab // 2 - 1)
            def _next():
                pltpu.async_copy(nbr_rows(2 * g2 + 2), slab_a, sem_a)

            eslab(2 * g2 + 1, slab_b, sem_b)
            return carry

        lax.fori_loop(0, nslab // 2, spair, 0)
        pltpu.sync_copy(eout_v, eblk_h.at[pl.ds(wid * (APT // 8), APT // 8)])

        def body(v, accs):
            acc_o, acc_c = accs
            an = plsc.load_gather(anum_v, [cai_v[pl.ds(v * 16, 16)]])
            one = jnp.ones((16,), jnp.int32)
            zero = jnp.zeros((16,), jnp.int32)
            is_c = an == 6
            is_o = jnp.logical_and(an != 6, an != 1)
            return (acc_o + jnp.where(is_o, one, zero),
                    acc_c + jnp.where(is_c, one, zero))

        acc_o, acc_c = lax.fori_loop(
            0, APT // 16, body,
            (jnp.zeros((16,), jnp.int32), jnp.zeros((16,), jnp.int32)))
        so = jnp.sum(acc_o)
        sc = jnp.sum(acc_c)
        res = jnp.where(iota == 0, jnp.full((16,), so, jnp.int32),
                        jnp.where(iota == 1, jnp.full((16,), sc, jnp.int32),
                                  jnp.zeros((16,), jnp.int32)))
        res_v[...] = res
        pltpu.sync_copy(res_v, cnt_h.at[pl.ds(wid * 16, 16)])

    return count_kernel


@functools.lru_cache(maxsize=1)
def _make_main_kernel():
    mesh = plsc.VectorSubcoreMesh(core_axis_name="c", subcore_axis_name="s",
                                  num_cores=2, num_subcores=16)

    @functools.partial(
        pl.kernel,
        out_type=jax.ShapeDtypeStruct((B, POS_PAD, HID_DIM), jnp.float32),
        mesh=mesh,
        compiler_params=pltpu.CompilerParams(needs_layout_passes=False),
        scratch_types=[
            pltpu.VMEM((N_ATOMS,), jnp.int32),
            pltpu.VMEM((NTYPE, NBR_FEA_LEN), jnp.float32),
            pltpu.VMEM((NTYPE, NBR_FEA_LEN), jnp.float32),
            pltpu.VMEM((32,), jnp.int32),
            pltpu.VMEM((160,), jnp.int32),
            pltpu.VMEM((16,), jnp.int32),
            pltpu.VMEM((16,), jnp.int32),
            pltpu.VMEM((16,), jnp.int32),
            pltpu.VMEM((16,), jnp.int32),
            pltpu.VMEM((CH, 128), jnp.int32),
            pltpu.VMEM((CH, 128), jnp.int32),
            pltpu.VMEM((CH, HID_DIM), jnp.float32),
            pltpu.VMEM((CH, HID_DIM), jnp.float32),
            pltpu.SemaphoreType.DMA,
            pltpu.SemaphoreType.DMA,
            pltpu.SemaphoreType.DMA,
            pltpu.SemaphoreType.DMA,
        ],
    )
    def main_kernel(cai_h, eblk_h, fea_h, node_h, edge_h, pos_h, nn_h,
                    out_h, cai_v, node_v, edge_v, nn_v, posall_v,
                    aidx_a, arow_a, aidx_b, arow_b, eb_a, eb_b, buf_a, buf_b,
                    semf_a, seme_a, semf_b, seme_b):
        wid = lax.axis_index("s") * 2 + lax.axis_index("c")
        pltpu.sync_copy(cai_h, cai_v)
        pltpu.sync_copy(node_h, node_v)
        pltpu.sync_copy(edge_h, edge_v)
        pltpu.sync_copy(nn_h, nn_v)
        pltpu.sync_copy(pos_h.at[pl.ds(wid * 160, 160)], posall_v)

        iota = lax.iota(jnp.int32, 16)
        nchunks = jnp.where(wid < NCHUNK - (NCHUNK // NW) * NW,
                            NCHUNK // NW + 1, NCHUNK // NW)

        slots = ((aidx_a, arow_a, eb_a, buf_a, semf_a, seme_a),
                 (aidx_b, arow_b, eb_b, buf_b, semf_b, seme_b))

        def issue(k, slot):
            aidx_v, arow_v, eb_v, buf_v, semf, seme = slot
            c = wid + k * NW
            bi = lax.div(c, CPC)
            a = plsc.load_gather(cai_v, [bi * L + posall_v[pl.ds(k * CH, CH)]])
            aidx_v[...] = a
            arow_v[...] = lax.shift_right_logical(a, 3)
            pltpu.async_copy(fea_h.at[aidx_v], buf_v, semf)
            pltpu.async_copy(eblk_h.at[arow_v], eb_v, seme)

        def process(k, slot):
            aidx_v, arow_v, eb_v, buf_v, semf, seme = slot
            c = wid + k * NW
            bi = lax.div(c, CPC)
            jc = c - bi * CPC
            j0 = jc * CH
            bi_spl = jnp.full((16,), bi, jnp.int32)
            no = plsc.load_gather(nn_v, [bi_spl])
            nc = plsc.load_gather(nn_v, [bi_spl + 16])
            pltpu.make_async_copy(fea_h.at[aidx_v], buf_v, semf).wait()
            pltpu.make_async_copy(eblk_h.at[arow_v], eb_v, seme).wait()

            def row_body(r, carry2):
                r_spl = jnp.full((16,), r, jnp.int32)
                a_spl = plsc.load_gather(aidx_v, [r_spl])
                ecol = (a_spl & 7) * 16
                na_spl = plsc.load_gather(eb_v, [r_spl, ecol + MAX_NBR])
                j_spl = jnp.full((16,), j0 + r, jnp.int32)
                ok = jnp.where(j_spl < 180, j_spl < no, j_spl - 180 < nc)
                val_spl = jnp.where(ok, jnp.full((16,), 1.0, jnp.float32),
                                    jnp.zeros((16,), jnp.float32))
                nd = [plsc.load_gather(node_v, [na_spl, iota + q * 16])
                      for q in range(4)]
                for m in range(MAX_NBR):
                    ei_spl = plsc.load_gather(eb_v, [r_spl, ecol + m])
                    for q in range(4):
                        ed = plsc.load_gather(edge_v, [ei_spl, iota + q * 16])
                        fe = buf_v[r, pl.ds(m * NBR_FEA_LEN + q * 16, 16)]
                        buf_v[r, pl.ds(m * NBR_FEA_LEN + q * 16, 16)] = (
                            (nd[q] + ed + fe) * val_spl)
                return carry2

            lax.fori_loop(0, CH, row_body, 0)
            pltpu.sync_copy(buf_v, out_h.at[bi, pl.ds(j0, CH)])

        issue(0, slots[0])

        def pair_body(k2, carry):
            k_b = 2 * k2 + 1
            k_a2 = 2 * k2 + 2

            @pl.when(k_b < nchunks)
            def _issue_b():
                issue(k_b, slots[1])

            process(2 * k2, slots[0])

            @pl.when(k_a2 < nchunks)
            def _issue_a():
                issue(k_a2, slots[0])

            @pl.when(k_b < nchunks)
            def _process_b():
                process(k_b, slots[1])

            return carry

        lax.fori_loop(0, (NCHUNK // NW + 2) // 2, pair_body, 0)

    return main_kernel


def kernel(atom_num, nbr_idx, nbr_fea, crystal_atom_idx, node_table, edge_table):
    atom_num = atom_num.astype(jnp.int32)
    nbr_idx = nbr_idx.astype(jnp.int32)
    cai_flat = crystal_atom_idx.astype(jnp.int32).reshape(-1)

    cnts, eblk = _make_count_kernel()(atom_num, cai_flat, nbr_idx)
    parts = cnts.reshape(NW, 16)
    n_others = parts[0::2, 0] + parts[1::2, 0]
    n_carbon = parts[0::2, 1] + parts[1::2, 1]

    pos = _build_positions(n_others, n_carbon)
    nn = jnp.concatenate([n_others, n_carbon]).astype(jnp.int32)

    pos_chunks = jnp.pad(pos.reshape(NCHUNK, CH), ((0, 320 - NCHUNK), (0, 0)))
    pos_wm = pos_chunks.reshape(10, NW, CH).transpose(1, 0, 2).reshape(-1)

    fea2 = nbr_fea.reshape(N_ATOMS, HID_DIM)
    out = _make_main_kernel()(cai_flat, eblk, fea2, node_table, edge_table,
                              pos_wm, nn)
    graph_emb = out[:, :MAX_GRAPH_LEN]
    mask = (graph_emb.sum(axis=-1) != 0).astype(jnp.float32)
    return graph_emb, mask
